# Initial kernel scaffold; baseline (speedup 1.0000x reference)
import jax
import jax.numpy as jnp
from jax.experimental import pallas as pl


def kernel(x, edge_index, W1, b1, W2, b2, W3, b3, W4, b4):
    raise NotImplementedError("write your pallas kernel here")


# speedup vs baseline: 11.1853x; 11.1853x over previous
import functools

import jax
import jax.numpy as jnp
from jax import lax
from jax.experimental import pallas as pl
from jax.experimental.pallas import tpu as pltpu
from jax.experimental.pallas import tpu_sc as plsc

N = 50000
E = 800000
B = 128
NSUB = 16
NCORE = 2
N_ACC = 50048
NZ = N_ACC // NSUB
CH_HALF = -(-(E // (NCORE * NSUB)) // B)
CH_FULL = -(-(E // NSUB) // B)

RB = 2000
GRID = N // RB


@functools.lru_cache(maxsize=None)
def _make_agg(ch, w):
    mesh = plsc.VectorSubcoreMesh(core_axis_name="c", subcore_axis_name="s")

    @functools.partial(
        pl.kernel,
        out_type=jax.ShapeDtypeStruct((NCORE, N_ACC, w), jnp.float32),
        mesh=mesh,
        scratch_types=[
            pltpu.VMEM((B,), jnp.int32),
            pltpu.VMEM((B,), jnp.int32),
            pltpu.VMEM((B, w), jnp.float32),
            pltpu.VMEM_SHARED((N_ACC, w), jnp.float32),
            pltpu.SemaphoreType.DMA,
        ],
        compiler_params=pltpu.CompilerParams(use_tc_tiling_on_sc=False),
    )
    def agg(table_hbm, sidx_hbm, didx_hbm, zeros_hbm, out_hbm,
            sidx, didx, rows, acc, sem):
        c = lax.axis_index("c")
        s = lax.axis_index("s")
        pltpu.sync_copy(zeros_hbm, acc.at[pl.ds(s * NZ, NZ)])
        plsc.subcore_barrier()

        @pl.loop(0, ch)
        def _(k):
            pltpu.sync_copy(sidx_hbm.at[c, s, k], sidx)
            pltpu.sync_copy(didx_hbm.at[c, s, k], didx)
            pltpu.async_copy(table_hbm.at[sidx], rows, sem).wait()
            pltpu.sync_copy(rows, acc.at[didx], add=True)

        plsc.subcore_barrier()
        pltpu.sync_copy(acc.at[pl.ds(s * NZ, NZ)],
                        out_hbm.at[c, pl.ds(s * NZ, NZ)])

    return agg


def _agg(table, sidx, didx, w):
    ch = sidx.shape[2]
    zeros = jnp.zeros((NZ, w), jnp.float32)
    return _make_agg(ch, w)(table, sidx, didx, zeros)


def _pack_half(a, pad_val):
    per = E // (NCORE * NSUB)
    m = a.reshape(NCORE * NSUB, per)
    m = jnp.pad(m, ((0, 0), (0, CH_HALF * B - per)), constant_values=pad_val)
    return m.reshape(NCORE, NSUB, CH_HALF, B)


def _pack_full(a, pad_val):
    per = E // NSUB
    m = a.reshape(NSUB, per)
    m = jnp.pad(m, ((0, 0), (0, CH_FULL * B - per)), constant_values=pad_val)
    return m.reshape(NSUB, CH_FULL, B)


def _row_spec(w):
    return pl.BlockSpec((RB, w), lambda i: (i, 0))


def _fix_spec(shape):
    return pl.BlockSpec(shape, lambda i: (0,) * len(shape))


def _s0_body(dp0_ref, dp1_ref, x_ref, t1_ref, dinv8_ref):
    deg = dp0_ref[:, 0:1] + dp1_ref[:, 0:1] + 1.0
    di = lax.rsqrt(deg)
    dinv8_ref[...] = jnp.broadcast_to(di, (RB, 8))
    t = di * x_ref[...]
    t1_ref[...] = jnp.concatenate([t, jnp.zeros((RB, 2), jnp.float32)], axis=1)


def _stage0(dp0, dp1, x):
    return pl.pallas_call(
        _s0_body,
        grid=(GRID,),
        in_specs=[_row_spec(8), _row_spec(8), _row_spec(6)],
        out_specs=[_row_spec(8), _row_spec(8)],
        out_shape=[jax.ShapeDtypeStruct((N, 8), jnp.float32),
                   jax.ShapeDtypeStruct((N, 8), jnp.float32)],
    )(dp0, dp1, x)


def _layer_body(a0_ref, a1_ref, tp_ref, dinv8_ref, w_ref, b_ref, out_ref):
    di = dinv8_ref[:, 0:1]
    u = (a0_ref[...] + a1_ref[...] + tp_ref[...]) * di
    h = jnp.dot(u, w_ref[...], preferred_element_type=jnp.float32) + b_ref[...]
    out_ref[...] = jnp.maximum(h, 0.0) * di


def _layer(a0, a1, tp, dinv8, w_mat, b, wi, wo):
    return pl.pallas_call(
        _layer_body,
        grid=(GRID,),
        in_specs=[_row_spec(wi), _row_spec(wi), _row_spec(wi), _row_spec(8),
                  _fix_spec((wi, wo)), _fix_spec((wo,))],
        out_specs=_row_spec(wo),
        out_shape=jax.ShapeDtypeStruct((N, wo), jnp.float32),
    )(a0, a1, tp, dinv8, w_mat, b)


def _s3_body(al_ref, ah_ref, t3_ref, dinv8_ref, w3_ref, b3_ref, w4_ref,
             t4_ref):
    di = dinv8_ref[:, 0:1]
    a = jnp.concatenate([al_ref[...], ah_ref[...]], axis=1)
    u = (a + t3_ref[...]) * di
    h = jnp.dot(u, w3_ref[...], preferred_element_type=jnp.float32) + b3_ref[...]
    h = jnp.maximum(h, 0.0)
    z = jnp.dot(h, w4_ref[...], preferred_element_type=jnp.float32) * di
    t4_ref[...] = jnp.concatenate([z, jnp.zeros((RB, 6), jnp.float32)], axis=1)


def _stage3(a_lo, a_hi, t3, dinv8, w3, b3, w4):
    return pl.pallas_call(
        _s3_body,
        grid=(GRID,),
        in_specs=[_row_spec(32), _row_spec(32), _row_spec(64), _row_spec(8),
                  _fix_spec((64, 128)), _fix_spec((128,)), _fix_spec((128, 2))],
        out_specs=_row_spec(8),
        out_shape=jax.ShapeDtypeStruct((N, 8), jnp.float32),
    )(a_lo, a_hi, t3, dinv8, w3, b3, w4)


def _s4_body(a0_ref, a1_ref, t4_ref, dinv8_ref, b4_ref, out_ref):
    di = dinv8_ref[:, 0:1]
    v = (a0_ref[...] + a1_ref[...] + t4_ref[...])[:, 0:2] * di + b4_ref[...]
    m = jnp.max(v, axis=1, keepdims=True)
    e = jnp.exp(v - m)
    out_ref[...] = (v - m) - jnp.log(jnp.sum(e, axis=1, keepdims=True))


def _stage4(a0, a1, t4, dinv8, b4):
    return pl.pallas_call(
        _s4_body,
        grid=(GRID,),
        in_specs=[_row_spec(8), _row_spec(8), _row_spec(8), _row_spec(8),
                  _fix_spec((2,))],
        out_specs=_row_spec(2),
        out_shape=jax.ShapeDtypeStruct((N, 2), jnp.float32),
    )(a0, a1, t4, dinv8, b4)


def _pad_rows(t):
    return jnp.pad(t, ((0, 8), (0, 0)))


def kernel(x, edge_index, W1, b1, W2, b2, W3, b3, W4, b4):
    src = edge_index[0].astype(jnp.int32)
    dst = edge_index[1].astype(jnp.int32)

    se = _pack_half(src, N)
    de = _pack_half(dst, N)
    s16 = src.reshape(NSUB, E // NSUB)
    d16 = dst.reshape(NSUB, E // NSUB)
    padw = CH_FULL * B - E // NSUB
    sc = jnp.stack([
        jnp.pad(s16, ((0, 0), (0, padw)), constant_values=2 * N),
        jnp.pad(s16 + N, ((0, 0), (0, padw)), constant_values=2 * N),
    ]).reshape(NCORE, NSUB, CH_FULL, B)
    dc = jnp.broadcast_to(
        jnp.pad(d16, ((0, 0), (0, padw)),
                constant_values=N).reshape(1, NSUB, CH_FULL, B),
        (NCORE, NSUB, CH_FULL, B))

    ones_tab = jnp.ones((N + 8, 8), jnp.float32)
    dp = _agg(ones_tab, se, de, 8)
    t1, dinv8 = _stage0(dp[0], dp[1], x)

    a1 = _agg(_pad_rows(t1), se, de, 8)
    t2 = _layer(a1[0], a1[1], t1, dinv8, jnp.pad(W1, ((0, 2), (0, 0))), b1,
                8, 32)

    a2 = _agg(_pad_rows(t2), se, de, 32)
    t3 = _layer(a2[0], a2[1], t2, dinv8, W2, b2, 32, 64)

    tab3 = jnp.concatenate(
        [t3[:, :32], t3[:, 32:], jnp.zeros((8, 32), jnp.float32)], axis=0)
    a3 = _agg(tab3, sc, dc, 32)
    t4 = _stage3(a3[0], a3[1], t3, dinv8, W3, b3, W4)

    a4 = _agg(_pad_rows(t4), se, de, 8)
    return _stage4(a4[0], a4[1], t4, dinv8, b4)


# speedup vs baseline: 18.7312x; 1.6746x over previous
import functools

import jax
import jax.numpy as jnp
from jax import lax
from jax.experimental import pallas as pl
from jax.experimental.pallas import tpu as pltpu
from jax.experimental.pallas import tpu_sc as plsc

N = 50000
E = 800000
B = 128
NSUB = 16
NCORE = 2
N_ACC = 50048
NZ = N_ACC // NSUB
NCH = E // B
CH_HALF = -(-(NCH // NCORE) // NSUB)
CH_FULL = -(-NCH // NSUB)
G = 64
NCH_PAD = 6400

RB = 2000
GRID = N // RB


@functools.lru_cache(maxsize=None)
def _make_agg(w, full):
    mesh = plsc.VectorSubcoreMesh(core_axis_name="c", subcore_axis_name="s")
    per = NCH // NCORE if not full else NCH
    lo = per // NSUB
    extra = per - lo * NSUB
    ngrp = -(-(lo + 1) // G)

    @functools.partial(
        pl.kernel,
        out_type=jax.ShapeDtypeStruct((NCORE, N_ACC, w), jnp.float32),
        mesh=mesh,
        scratch_types=[
            pltpu.VMEM((G, B), jnp.int32),
            pltpu.VMEM((G, B), jnp.int32),
            pltpu.VMEM((B, w), jnp.float32),
            pltpu.VMEM((B, w), jnp.float32),
            pltpu.VMEM_SHARED((N_ACC, w), jnp.float32),
            pltpu.SemaphoreType.DMA,
            pltpu.SemaphoreType.DMA,
        ],
        compiler_params=pltpu.CompilerParams(use_tc_tiling_on_sc=False),
    )
    def agg(table_hbm, sidx_hbm, didx_hbm, zeros_hbm, out_hbm,
            sidx, didx, rows0, rows1, acc, sem0, sem1):
        c = lax.axis_index("c")
        s = lax.axis_index("s")
        nch = lo + jnp.where(s < extra, 1, 0)
        base = s * lo + jnp.minimum(s, extra) + (0 if full else c * per)
        csel = c if full else 0

        pltpu.sync_copy(zeros_hbm, acc.at[pl.ds(s * NZ, NZ)])
        plsc.subcore_barrier()

        def step(j, cnt, cur_rows, cur_sem, nxt_rows, nxt_sem):
            pltpu.make_async_copy(
                table_hbm.at[pl.ds(0, B)], cur_rows, cur_sem).wait()

            @pl.when(j + 1 < cnt)
            def _():
                pltpu.async_copy(
                    table_hbm.at[sidx.at[j + 1]], nxt_rows, nxt_sem)

            pltpu.sync_copy(cur_rows, acc.at[didx.at[j]], add=True)

        @pl.loop(0, ngrp)
        def _(g):
            cnt = jnp.minimum(G, nch - g * G)

            @pl.when(cnt > 0)
            def _():
                pltpu.sync_copy(sidx_hbm.at[csel, pl.ds(base + g * G, G)],
                                sidx)
                pltpu.sync_copy(didx_hbm.at[0, pl.ds(base + g * G, G)], didx)
                pltpu.async_copy(table_hbm.at[sidx.at[0]], rows0, sem0)

                @pl.loop(0, cnt)
                def _(j):
                    @pl.when(lax.rem(j, 2) == 0)
                    def _():
                        step(j, cnt, rows0, sem0, rows1, sem1)

                    @pl.when(lax.rem(j, 2) == 1)
                    def _():
                        step(j, cnt, rows1, sem1, rows0, sem0)

        plsc.subcore_barrier()
        pltpu.sync_copy(acc.at[pl.ds(s * NZ, NZ)],
                        out_hbm.at[c, pl.ds(s * NZ, NZ)])

    return agg


def _agg(table, sidx, didx, w, full=False):
    zeros = jnp.zeros((NZ, w), jnp.float32)
    return _make_agg(w, full)(table, sidx, didx, zeros)


def _row_spec(w):
    return pl.BlockSpec((RB, w), lambda i: (i, 0))


def _fix_spec(shape):
    return pl.BlockSpec(shape, lambda i: (0,) * len(shape))


def _s0_body(dp0_ref, dp1_ref, x_ref, t1_ref, dinv8_ref):
    deg = dp0_ref[:, 0:1] + dp1_ref[:, 0:1] + 1.0
    di = lax.rsqrt(deg)
    dinv8_ref[...] = jnp.broadcast_to(di, (RB, 8))
    t = di * x_ref[...]
    t1_ref[...] = jnp.concatenate([t, jnp.zeros((RB, 2), jnp.float32)], axis=1)


def _stage0(dp0, dp1, x):
    return pl.pallas_call(
        _s0_body,
        grid=(GRID,),
        in_specs=[_row_spec(8), _row_spec(8), _row_spec(6)],
        out_specs=[_row_spec(8), _row_spec(8)],
        out_shape=[jax.ShapeDtypeStruct((N, 8), jnp.float32),
                   jax.ShapeDtypeStruct((N, 8), jnp.float32)],
    )(dp0, dp1, x)


def _layer_body(a0_ref, a1_ref, tp_ref, dinv8_ref, w_ref, b_ref, out_ref):
    di = dinv8_ref[:, 0:1]
    u = (a0_ref[...] + a1_ref[...] + tp_ref[...]) * di
    h = jnp.dot(u, w_ref[...], preferred_element_type=jnp.float32) + b_ref[...]
    out_ref[...] = jnp.maximum(h, 0.0) * di


def _layer(a0, a1, tp, dinv8, w_mat, b, wi, wo):
    return pl.pallas_call(
        _layer_body,
        grid=(GRID,),
        in_specs=[_row_spec(wi), _row_spec(wi), _row_spec(wi), _row_spec(8),
                  _fix_spec((wi, wo)), _fix_spec((wo,))],
        out_specs=_row_spec(wo),
        out_shape=jax.ShapeDtypeStruct((N, wo), jnp.float32),
    )(a0, a1, tp, dinv8, w_mat, b)


def _s3_body(al_ref, ah_ref, t3_ref, dinv8_ref, w3_ref, b3_ref, w4_ref,
             t4_ref):
    di = dinv8_ref[:, 0:1]
    a = jnp.concatenate([al_ref[...], ah_ref[...]], axis=1)
    u = (a + t3_ref[...]) * di
    h = jnp.dot(u, w3_ref[...], preferred_element_type=jnp.float32) + b3_ref[...]
    h = jnp.maximum(h, 0.0)
    z = jnp.dot(h, w4_ref[...], preferred_element_type=jnp.float32) * di
    t4_ref[...] = jnp.concatenate([z, jnp.zeros((RB, 6), jnp.float32)], axis=1)


def _stage3(a_lo, a_hi, t3, dinv8, w3, b3, w4):
    return pl.pallas_call(
        _s3_body,
        grid=(GRID,),
        in_specs=[_row_spec(32), _row_spec(32), _row_spec(64), _row_spec(8),
                  _fix_spec((64, 128)), _fix_spec((128,)), _fix_spec((128, 2))],
        out_specs=_row_spec(8),
        out_shape=jax.ShapeDtypeStruct((N, 8), jnp.float32),
    )(a_lo, a_hi, t3, dinv8, w3, b3, w4)


def _s4_body(a0_ref, a1_ref, t4_ref, dinv8_ref, b4_ref, out_ref):
    di = dinv8_ref[:, 0:1]
    v = (a0_ref[...] + a1_ref[...] + t4_ref[...])[:, 0:2] * di + b4_ref[...]
    m = jnp.max(v, axis=1, keepdims=True)
    e = jnp.exp(v - m)
    out_ref[...] = (v - m) - jnp.log(jnp.sum(e, axis=1, keepdims=True))


def _stage4(a0, a1, t4, dinv8, b4):
    return pl.pallas_call(
        _s4_body,
        grid=(GRID,),
        in_specs=[_row_spec(8), _row_spec(8), _row_spec(8), _row_spec(8),
                  _fix_spec((2,))],
        out_specs=_row_spec(2),
        out_shape=jax.ShapeDtypeStruct((N, 2), jnp.float32),
    )(a0, a1, t4, dinv8, b4)


def _pad_rows(t):
    return jnp.pad(t, ((0, 8), (0, 0)))


def kernel(x, edge_index, W1, b1, W2, b2, W3, b3, W4, b4):
    src = edge_index[0].astype(jnp.int32)
    dst = edge_index[1].astype(jnp.int32)

    sp = jnp.pad(src.reshape(NCH, B), ((0, NCH_PAD - NCH), (0, 0)))
    sidx_all = jnp.stack([sp, sp + N])
    didx_all = jnp.pad(dst.reshape(NCH, B),
                       ((0, NCH_PAD - NCH), (0, 0)))[None]

    ones_tab = jnp.ones((N + 8, 8), jnp.float32)
    dp = _agg(ones_tab, sidx_all, didx_all, 8)
    t1, dinv8 = _stage0(dp[0], dp[1], x)

    a1 = _agg(_pad_rows(t1), sidx_all, didx_all, 8)
    t2 = _layer(a1[0], a1[1], t1, dinv8, jnp.pad(W1, ((0, 2), (0, 0))), b1,
                8, 32)

    a2 = _agg(_pad_rows(t2), sidx_all, didx_all, 32)
    t3 = _layer(a2[0], a2[1], t2, dinv8, W2, b2, 32, 64)

    tab3 = jnp.concatenate(
        [t3[:, :32], t3[:, 32:], jnp.zeros((8, 32), jnp.float32)], axis=0)
    a3 = _agg(tab3, sidx_all, didx_all, 32, full=True)
    t4 = _stage3(a3[0], a3[1], t3, dinv8, W3, b3, W4)

    a4 = _agg(_pad_rows(t4), sidx_all, didx_all, 8)
    return _stage4(a4[0], a4[1], t4, dinv8, b4)


# speedup vs baseline: 29.1548x; 1.5565x over previous
import functools

import jax
import jax.numpy as jnp
from jax import lax
from jax.experimental import pallas as pl
from jax.experimental.pallas import tpu as pltpu
from jax.experimental.pallas import tpu_sc as plsc

N = 50000
E = 800000
B = 128
NSUB = 16
NCORE = 2
N_ACC = 50048
NZ = N_ACC // NSUB
NCH = E // B
CH_HALF = -(-(NCH // NCORE) // NSUB)
CH_FULL = -(-NCH // NSUB)
G = 48
NCH_PAD = 6400

RB = 2000
GRID = N // RB


@functools.lru_cache(maxsize=None)
def _make_agg(w, full):
    mesh = plsc.VectorSubcoreMesh(core_axis_name="c", subcore_axis_name="s")
    per = NCH // NCORE if not full else NCH
    lo = per // NSUB
    extra = per - lo * NSUB
    ngrp = -(-(lo + 1) // G)

    @functools.partial(
        pl.kernel,
        out_type=jax.ShapeDtypeStruct((NCORE, N_ACC, w), jnp.float32),
        mesh=mesh,
        scratch_types=[
            pltpu.VMEM((G, B), jnp.int32),
            pltpu.VMEM((G, B), jnp.int32),
            [pltpu.VMEM((B, w), jnp.float32)] * 4,
            pltpu.VMEM_SHARED((N_ACC, w), jnp.float32),
            [pltpu.SemaphoreType.DMA] * 4,
            [pltpu.SemaphoreType.DMA] * 4,
        ],
        compiler_params=pltpu.CompilerParams(use_tc_tiling_on_sc=False),
    )
    def agg(table_hbm, sidx_hbm, didx_hbm, zeros_hbm, out_hbm,
            sidx, didx, rows, acc, gsem, ssem):
        c = lax.axis_index("c")
        s = lax.axis_index("s")
        nch = lo + jnp.where(s < extra, 1, 0)
        base = s * lo + jnp.minimum(s, extra) + (0 if full else c * per)
        csel = c if full else 0

        pltpu.sync_copy(zeros_hbm, acc.at[pl.ds(s * NZ, NZ)])
        plsc.subcore_barrier()

        def step(k, cnt, p):
            q = (p + 3) % 4
            pltpu.make_async_copy(
                table_hbm.at[pl.ds(0, B)], rows[p], gsem[p]).wait()
            pltpu.async_copy(rows[p], acc.at[didx.at[k]], ssem[p], add=True)

            @pl.when(k >= 1)
            def _():
                pltpu.make_async_copy(
                    rows[q], acc.at[didx.at[k - 1]], ssem[q]).wait()

            @pl.when(k + 3 < cnt)
            def _():
                pltpu.async_copy(
                    table_hbm.at[sidx.at[k + 3]], rows[q], gsem[q])

        @pl.loop(0, ngrp)
        def _(g):
            cnt = jnp.minimum(G, nch - g * G)

            @pl.when(cnt > 0)
            def _():
                pltpu.sync_copy(sidx_hbm.at[csel, pl.ds(base + g * G, G)],
                                sidx)
                pltpu.sync_copy(didx_hbm.at[0, pl.ds(base + g * G, G)], didx)
                for r in range(3):
                    @pl.when(r < cnt)
                    def _():
                        pltpu.async_copy(
                            table_hbm.at[sidx.at[r]], rows[r], gsem[r])

                @pl.loop(0, cnt)
                def _(k):
                    for p in range(4):
                        @pl.when(lax.rem(k, 4) == p)
                        def _():
                            step(k, cnt, p)

                for p in range(4):
                    @pl.when(lax.rem(cnt - 1, 4) == p)
                    def _():
                        pltpu.make_async_copy(
                            rows[p], acc.at[didx.at[cnt - 1]],
                            ssem[p]).wait()

        plsc.subcore_barrier()
        pltpu.sync_copy(acc.at[pl.ds(s * NZ, NZ)],
                        out_hbm.at[c, pl.ds(s * NZ, NZ)])

    return agg


def _agg(table, sidx, didx, w, full=False):
    zeros = jnp.zeros((NZ, w), jnp.float32)
    return _make_agg(w, full)(table, sidx, didx, zeros)


def _row_spec(w):
    return pl.BlockSpec((RB, w), lambda i: (i, 0))


def _fix_spec(shape):
    return pl.BlockSpec(shape, lambda i: (0,) * len(shape))


def _s0_body(dp0_ref, dp1_ref, x_ref, t1_ref, dinv8_ref):
    deg = dp0_ref[:, 0:1] + dp1_ref[:, 0:1] + 1.0
    di = lax.rsqrt(deg)
    dinv8_ref[...] = jnp.broadcast_to(di, (RB, 8))
    t = di * x_ref[...]
    t1_ref[...] = jnp.concatenate([t, jnp.zeros((RB, 2), jnp.float32)], axis=1)


def _stage0(dp0, dp1, x):
    return pl.pallas_call(
        _s0_body,
        grid=(GRID,),
        in_specs=[_row_spec(8), _row_spec(8), _row_spec(6)],
        out_specs=[_row_spec(8), _row_spec(8)],
        out_shape=[jax.ShapeDtypeStruct((N, 8), jnp.float32),
                   jax.ShapeDtypeStruct((N, 8), jnp.float32)],
    )(dp0, dp1, x)


def _layer_body(a0_ref, a1_ref, tp_ref, dinv8_ref, w_ref, b_ref, out_ref):
    di = dinv8_ref[:, 0:1]
    u = (a0_ref[...] + a1_ref[...] + tp_ref[...]) * di
    h = jnp.dot(u, w_ref[...], preferred_element_type=jnp.float32) + b_ref[...]
    t = jnp.maximum(h, 0.0) * di
    if out_ref.shape[0] == 2:
        hw = out_ref.shape[2]
        out_ref[0] = t[:, :hw]
        out_ref[1] = t[:, hw:]
    else:
        out_ref[...] = t


def _layer(a0, a1, tp, dinv8, w_mat, b, wi, wo, split=False):
    if split:
        out_spec = pl.BlockSpec((2, RB, wo // 2), lambda i: (0, i, 0))
        out_shape = jax.ShapeDtypeStruct((2, N, wo // 2), jnp.float32)
    else:
        out_spec = _row_spec(wo)
        out_shape = jax.ShapeDtypeStruct((N, wo), jnp.float32)
    return pl.pallas_call(
        _layer_body,
        grid=(GRID,),
        in_specs=[_row_spec(wi), _row_spec(wi), _row_spec(wi), _row_spec(8),
                  _fix_spec((wi, wo)), _fix_spec((wo,))],
        out_specs=out_spec,
        out_shape=out_shape,
    )(a0, a1, tp, dinv8, w_mat, b)


def _s3_body(al_ref, ah_ref, t3_ref, dinv8_ref, w3_ref, b3_ref, w4_ref,
             t4_ref):
    di = dinv8_ref[:, 0:1]
    a = jnp.concatenate([al_ref[...], ah_ref[...]], axis=1)
    t3 = jnp.concatenate([t3_ref[0], t3_ref[1]], axis=1)
    u = (a + t3) * di
    h = jnp.dot(u, w3_ref[...], preferred_element_type=jnp.float32) + b3_ref[...]
    h = jnp.maximum(h, 0.0)
    z = jnp.dot(h, w4_ref[...], preferred_element_type=jnp.float32) * di
    t4_ref[...] = jnp.concatenate([z, jnp.zeros((RB, 6), jnp.float32)], axis=1)


def _stage3(a_lo, a_hi, t3, dinv8, w3, b3, w4):
    return pl.pallas_call(
        _s3_body,
        grid=(GRID,),
        in_specs=[_row_spec(32), _row_spec(32),
                  pl.BlockSpec((2, RB, 32), lambda i: (0, i, 0)), _row_spec(8),
                  _fix_spec((64, 128)), _fix_spec((128,)), _fix_spec((128, 2))],
        out_specs=_row_spec(8),
        out_shape=jax.ShapeDtypeStruct((N, 8), jnp.float32),
    )(a_lo, a_hi, t3, dinv8, w3, b3, w4)


def _s4_body(a0_ref, a1_ref, t4_ref, dinv8_ref, b4_ref, out_ref):
    di = dinv8_ref[:, 0:1]
    v = (a0_ref[...] + a1_ref[...] + t4_ref[...])[:, 0:2] * di + b4_ref[...]
    m = jnp.max(v, axis=1, keepdims=True)
    e = jnp.exp(v - m)
    out_ref[...] = (v - m) - jnp.log(jnp.sum(e, axis=1, keepdims=True))


def _stage4(a0, a1, t4, dinv8, b4):
    return pl.pallas_call(
        _s4_body,
        grid=(GRID,),
        in_specs=[_row_spec(8), _row_spec(8), _row_spec(8), _row_spec(8),
                  _fix_spec((2,))],
        out_specs=_row_spec(2),
        out_shape=jax.ShapeDtypeStruct((N, 2), jnp.float32),
    )(a0, a1, t4, dinv8, b4)


def kernel(x, edge_index, W1, b1, W2, b2, W3, b3, W4, b4):
    src = edge_index[0].astype(jnp.int32)
    dst = edge_index[1].astype(jnp.int32)

    sp = jnp.pad(src.reshape(NCH, B), ((0, NCH_PAD - NCH), (0, 0)))
    sidx_all = jnp.stack([sp, sp + N])
    didx_all = jnp.pad(dst.reshape(NCH, B),
                       ((0, NCH_PAD - NCH), (0, 0)))[None]

    ones_tab = jnp.ones((N, 8), jnp.float32)
    dp = _agg(ones_tab, sidx_all, didx_all, 8)
    t1, dinv8 = _stage0(dp[0], dp[1], x)

    a1 = _agg(t1, sidx_all, didx_all, 8)
    t2 = _layer(a1[0], a1[1], t1, dinv8, jnp.pad(W1, ((0, 2), (0, 0))), b1,
                8, 32)

    a2 = _agg(t2, sidx_all, didx_all, 32)
    t3s = _layer(a2[0], a2[1], t2, dinv8, W2, b2, 32, 64, split=True)

    a3 = _agg(t3s.reshape(2 * N, 32), sidx_all, didx_all, 32, full=True)
    t4 = _stage3(a3[0], a3[1], t3s, dinv8, W3, b3, W4)

    a4 = _agg(t4, sidx_all, didx_all, 8)
    return _stage4(a4[0], a4[1], t4, dinv8, b4)


# speedup vs baseline: 30.7528x; 1.0548x over previous
import functools

import jax
import jax.numpy as jnp
from jax import lax
from jax.experimental import pallas as pl
from jax.experimental.pallas import tpu as pltpu
from jax.experimental.pallas import tpu_sc as plsc

N = 50000
E = 800000
B = 128
NSUB = 16
NCORE = 2
N_ACC = 50048
NZ = N_ACC // NSUB
NCH = E // B
CH_HALF = -(-(NCH // NCORE) // NSUB)
CH_FULL = -(-NCH // NSUB)
G = 48
NCH_PAD = 6400

RB = 2000
GRID = N // RB


@functools.lru_cache(maxsize=None)
def _make_agg(w, full, hist=False):
    mesh = plsc.VectorSubcoreMesh(core_axis_name="c", subcore_axis_name="s")
    per = NCH // NCORE if not full else NCH
    lo = per // NSUB
    extra = per - lo * NSUB
    g = 128 if w <= 8 else G
    ngrp = -(-(lo + 1) // g)

    @functools.partial(
        pl.kernel,
        out_type=jax.ShapeDtypeStruct((NCORE, N_ACC, w), jnp.float32),
        mesh=mesh,
        scratch_types=[
            pltpu.VMEM((g, B), jnp.int32),
            pltpu.VMEM((g, B), jnp.int32),
            [pltpu.VMEM((B, w), jnp.float32)] * 4,
            pltpu.VMEM_SHARED((N_ACC, w), jnp.float32),
            [pltpu.SemaphoreType.DMA] * 4,
            [pltpu.SemaphoreType.DMA] * 4,
        ],
        compiler_params=pltpu.CompilerParams(use_tc_tiling_on_sc=False),
    )
    def agg(table_hbm, sidx_hbm, didx_hbm, zeros_hbm, out_hbm,
            sidx, didx, rows, acc, gsem, ssem):
        c = lax.axis_index("c")
        s = lax.axis_index("s")
        nch = lo + jnp.where(s < extra, 1, 0)
        base = s * lo + jnp.minimum(s, extra) + (0 if full else c * per)
        csel = c if full else 0

        pltpu.sync_copy(zeros_hbm, acc.at[pl.ds(s * NZ, NZ)])
        if hist:
            pltpu.sync_copy(table_hbm, rows[0])
        plsc.subcore_barrier()

        def step(k, cnt, p):
            q = (p + 3) % 4
            pltpu.make_async_copy(
                table_hbm.at[pl.ds(0, B)], rows[p], gsem[p]).wait()
            pltpu.async_copy(rows[p], acc.at[didx.at[k]], ssem[p], add=True)

            @pl.when(k >= 1)
            def _():
                pltpu.make_async_copy(
                    rows[q], acc.at[didx.at[k - 1]], ssem[q]).wait()

            @pl.when(k + 3 < cnt)
            def _():
                pltpu.async_copy(
                    table_hbm.at[sidx.at[k + 3]], rows[q], gsem[q])

        def hist_step(k, p):
            pltpu.async_copy(rows[0], acc.at[didx.at[k]], ssem[p], add=True)

            @pl.when(k >= 3)
            def _():
                pltpu.make_async_copy(
                    rows[0], acc.at[didx.at[k - 3]], ssem[(p + 1) % 4]).wait()

        @pl.loop(0, ngrp)
        def _(gi):
            cnt = jnp.minimum(g, nch - gi * g)

            @pl.when(cnt > 0)
            def _():
                pltpu.sync_copy(didx_hbm.at[0, pl.ds(base + gi * g, g)], didx)
                if hist:
                    @pl.loop(0, cnt)
                    def _(k):
                        for p in range(4):
                            @pl.when(lax.rem(k, 4) == p)
                            def _():
                                hist_step(k, p)

                    @pl.loop(jnp.maximum(cnt - 3, 0), cnt)
                    def _(r):
                        for p in range(4):
                            @pl.when(lax.rem(r, 4) == p)
                            def _():
                                pltpu.make_async_copy(
                                    rows[0], acc.at[didx.at[r]],
                                    ssem[p]).wait()
                else:
                    pltpu.sync_copy(
                        sidx_hbm.at[csel, pl.ds(base + gi * g, g)], sidx)
                    for r in range(3):
                        @pl.when(r < cnt)
                        def _():
                            pltpu.async_copy(
                                table_hbm.at[sidx.at[r]], rows[r], gsem[r])

                    @pl.loop(0, cnt)
                    def _(k):
                        for p in range(4):
                            @pl.when(lax.rem(k, 4) == p)
                            def _():
                                step(k, cnt, p)

                    for p in range(4):
                        @pl.when(lax.rem(cnt - 1, 4) == p)
                        def _():
                            pltpu.make_async_copy(
                                rows[p], acc.at[didx.at[cnt - 1]],
                                ssem[p]).wait()

        plsc.subcore_barrier()
        pltpu.sync_copy(acc.at[pl.ds(s * NZ, NZ)],
                        out_hbm.at[c, pl.ds(s * NZ, NZ)])

    return agg


def _agg(table, sidx, didx, w, full=False, hist=False):
    zeros = jnp.zeros((NZ, w), jnp.float32)
    return _make_agg(w, full, hist)(table, sidx, didx, zeros)


def _row_spec(w):
    return pl.BlockSpec((RB, w), lambda i: (i, 0))


def _fix_spec(shape):
    return pl.BlockSpec(shape, lambda i: (0,) * len(shape))


def _s0_body(dp0_ref, dp1_ref, x_ref, t1_ref, dinv8_ref):
    deg = dp0_ref[:, 0:1] + dp1_ref[:, 0:1] + 1.0
    di = lax.rsqrt(deg)
    dinv8_ref[...] = jnp.broadcast_to(di, (RB, 8))
    t = di * x_ref[...]
    t1_ref[...] = jnp.concatenate([t, jnp.zeros((RB, 2), jnp.float32)], axis=1)


def _stage0(dp0, dp1, x):
    return pl.pallas_call(
        _s0_body,
        grid=(GRID,),
        in_specs=[_row_spec(8), _row_spec(8), _row_spec(6)],
        out_specs=[_row_spec(8), _row_spec(8)],
        out_shape=[jax.ShapeDtypeStruct((N, 8), jnp.float32),
                   jax.ShapeDtypeStruct((N, 8), jnp.float32)],
    )(dp0, dp1, x)


def _layer_body(a0_ref, a1_ref, tp_ref, dinv8_ref, w_ref, b_ref, out_ref):
    di = dinv8_ref[:, 0:1]
    u = (a0_ref[...] + a1_ref[...] + tp_ref[...]) * di
    h = jnp.dot(u, w_ref[...], preferred_element_type=jnp.float32) + b_ref[...]
    t = jnp.maximum(h, 0.0) * di
    if out_ref.shape[0] == 2:
        hw = out_ref.shape[2]
        out_ref[0] = t[:, :hw]
        out_ref[1] = t[:, hw:]
    else:
        out_ref[...] = t


def _layer(a0, a1, tp, dinv8, w_mat, b, wi, wo, split=False):
    if split:
        out_spec = pl.BlockSpec((2, RB, wo // 2), lambda i: (0, i, 0))
        out_shape = jax.ShapeDtypeStruct((2, N, wo // 2), jnp.float32)
    else:
        out_spec = _row_spec(wo)
        out_shape = jax.ShapeDtypeStruct((N, wo), jnp.float32)
    return pl.pallas_call(
        _layer_body,
        grid=(GRID,),
        in_specs=[_row_spec(wi), _row_spec(wi), _row_spec(wi), _row_spec(8),
                  _fix_spec((wi, wo)), _fix_spec((wo,))],
        out_specs=out_spec,
        out_shape=out_shape,
    )(a0, a1, tp, dinv8, w_mat, b)


def _s3_body(al_ref, ah_ref, t3_ref, dinv8_ref, w3_ref, b3_ref, w4_ref,
             t4_ref):
    di = dinv8_ref[:, 0:1]
    a = jnp.concatenate([al_ref[...], ah_ref[...]], axis=1)
    t3 = jnp.concatenate([t3_ref[0], t3_ref[1]], axis=1)
    u = (a + t3) * di
    h = jnp.dot(u, w3_ref[...], preferred_element_type=jnp.float32) + b3_ref[...]
    h = jnp.maximum(h, 0.0)
    z = jnp.dot(h, w4_ref[...], preferred_element_type=jnp.float32) * di
    t4_ref[...] = jnp.concatenate([z, jnp.zeros((RB, 6), jnp.float32)], axis=1)


def _stage3(a_lo, a_hi, t3, dinv8, w3, b3, w4):
    return pl.pallas_call(
        _s3_body,
        grid=(GRID,),
        in_specs=[_row_spec(32), _row_spec(32),
                  pl.BlockSpec((2, RB, 32), lambda i: (0, i, 0)), _row_spec(8),
                  _fix_spec((64, 128)), _fix_spec((128,)), _fix_spec((128, 2))],
        out_specs=_row_spec(8),
        out_shape=jax.ShapeDtypeStruct((N, 8), jnp.float32),
    )(a_lo, a_hi, t3, dinv8, w3, b3, w4)


def _s4_body(a0_ref, a1_ref, t4_ref, dinv8_ref, b4_ref, out_ref):
    di = dinv8_ref[:, 0:1]
    v = (a0_ref[...] + a1_ref[...] + t4_ref[...])[:, 0:2] * di + b4_ref[...]
    m = jnp.max(v, axis=1, keepdims=True)
    e = jnp.exp(v - m)
    out_ref[...] = (v - m) - jnp.log(jnp.sum(e, axis=1, keepdims=True))


def _stage4(a0, a1, t4, dinv8, b4):
    return pl.pallas_call(
        _s4_body,
        grid=(GRID,),
        in_specs=[_row_spec(8), _row_spec(8), _row_spec(8), _row_spec(8),
                  _fix_spec((2,))],
        out_specs=_row_spec(2),
        out_shape=jax.ShapeDtypeStruct((N, 2), jnp.float32),
    )(a0, a1, t4, dinv8, b4)


def kernel(x, edge_index, W1, b1, W2, b2, W3, b3, W4, b4):
    src = edge_index[0].astype(jnp.int32)
    dst = edge_index[1].astype(jnp.int32)

    sp = jnp.pad(src.reshape(NCH, B), ((0, NCH_PAD - NCH), (0, 0)))
    sidx_all = jnp.stack([sp, sp + N])
    didx_all = jnp.pad(dst.reshape(NCH, B),
                       ((0, NCH_PAD - NCH), (0, 0)))[None]

    dp = _agg(jnp.ones((B, 8), jnp.float32), sidx_all, didx_all, 8, hist=True)
    t1, dinv8 = _stage0(dp[0], dp[1], x)

    a1 = _agg(t1, sidx_all, didx_all, 8)
    t2 = _layer(a1[0], a1[1], t1, dinv8, jnp.pad(W1, ((0, 2), (0, 0))), b1,
                8, 32)

    a2 = _agg(t2, sidx_all, didx_all, 32)
    t3s = _layer(a2[0], a2[1], t2, dinv8, W2, b2, 32, 64, split=True)

    a3 = _agg(t3s.reshape(2 * N, 32), sidx_all, didx_all, 32, full=True)
    t4 = _stage3(a3[0], a3[1], t3s, dinv8, W3, b3, W4)

    a4 = _agg(t4, sidx_all, didx_all, 8)
    return _stage4(a4[0], a4[1], t4, dinv8, b4)


# speedup vs baseline: 35.8850x; 1.1669x over previous
import functools

import jax
import jax.numpy as jnp
from jax import lax
from jax.experimental import pallas as pl
from jax.experimental.pallas import tpu as pltpu
from jax.experimental.pallas import tpu_sc as plsc

N = 50000
E = 800000
B = 128
NSUB = 16
NCORE = 2
N_ACC = 50048
NZ = N_ACC // NSUB
NCH = E // B
CH_HALF = -(-(NCH // NCORE) // NSUB)
CH_FULL = -(-NCH // NSUB)
G = 48
NCH_PAD = 6400

RB = 2000
GRID = N // RB


@functools.lru_cache(maxsize=None)
def _make_agg(w, full, hist=False):
    mesh = plsc.VectorSubcoreMesh(core_axis_name="c", subcore_axis_name="s")
    per = NCH // NCORE if not full else NCH
    lo = per // NSUB
    extra = per - lo * NSUB
    g = 128 if w <= 8 else G
    ngrp = -(-(lo + 1) // g)

    @functools.partial(
        pl.kernel,
        out_type=jax.ShapeDtypeStruct((NCORE, N_ACC, w), jnp.float32),
        mesh=mesh,
        scratch_types=[
            pltpu.VMEM((g, B), jnp.int32),
            pltpu.VMEM((g, B), jnp.int32),
            [pltpu.VMEM((B, w), jnp.float32)] * 4,
            pltpu.VMEM_SHARED((N_ACC, w), jnp.float32),
            [pltpu.SemaphoreType.DMA] * 4,
            [pltpu.SemaphoreType.DMA] * 4,
        ],
        compiler_params=pltpu.CompilerParams(use_tc_tiling_on_sc=False),
    )
    def agg(*refs):
        if hist:
            table_hbm, didx_hbm, zeros_hbm, out_hbm, \
                sidx, didx, rows, acc, gsem, ssem = refs
            sidx_hbm = None
        else:
            table_hbm, sidx_hbm, didx_hbm, zeros_hbm, out_hbm, \
                sidx, didx, rows, acc, gsem, ssem = refs
        c = lax.axis_index("c")
        s = lax.axis_index("s")
        nch = lo + jnp.where(s < extra, 1, 0)
        base = s * lo + jnp.minimum(s, extra) + (0 if full else c * per)
        csel = c if full else 0

        pltpu.sync_copy(zeros_hbm, acc.at[pl.ds(s * NZ, NZ)])
        if hist:
            pltpu.sync_copy(table_hbm, rows[0])
        plsc.subcore_barrier()

        def step(k, cnt, p):
            q = (p + 3) % 4
            pltpu.make_async_copy(
                table_hbm.at[pl.ds(0, B)], rows[p], gsem[p]).wait()
            pltpu.async_copy(rows[p], acc.at[didx.at[k]], ssem[p], add=True)

            @pl.when(k >= 1)
            def _():
                pltpu.make_async_copy(
                    rows[q], acc.at[didx.at[k - 1]], ssem[q]).wait()

            @pl.when(k + 3 < cnt)
            def _():
                pltpu.async_copy(
                    table_hbm.at[sidx.at[k + 3]], rows[q], gsem[q])

        def hist_step(k, p):
            pltpu.async_copy(rows[0], acc.at[didx.at[k]], ssem[p], add=True)

            @pl.when(k >= 3)
            def _():
                pltpu.make_async_copy(
                    rows[0], acc.at[didx.at[k - 3]], ssem[(p + 1) % 4]).wait()

        @pl.loop(0, ngrp)
        def _(gi):
            cnt = jnp.minimum(g, nch - gi * g)

            @pl.when(cnt > 0)
            def _():
                pltpu.sync_copy(didx_hbm.at[0, pl.ds(base + gi * g, g)], didx)
                if hist:
                    @pl.loop(0, cnt)
                    def _(k):
                        for p in range(4):
                            @pl.when(lax.rem(k, 4) == p)
                            def _():
                                hist_step(k, p)

                    @pl.loop(jnp.maximum(cnt - 3, 0), cnt)
                    def _(r):
                        for p in range(4):
                            @pl.when(lax.rem(r, 4) == p)
                            def _():
                                pltpu.make_async_copy(
                                    rows[0], acc.at[didx.at[r]],
                                    ssem[p]).wait()
                else:
                    pltpu.sync_copy(
                        sidx_hbm.at[csel, pl.ds(base + gi * g, g)], sidx)
                    for r in range(3):
                        @pl.when(r < cnt)
                        def _():
                            pltpu.async_copy(
                                table_hbm.at[sidx.at[r]], rows[r], gsem[r])

                    @pl.loop(0, cnt)
                    def _(k):
                        for p in range(4):
                            @pl.when(lax.rem(k, 4) == p)
                            def _():
                                step(k, cnt, p)

                    for p in range(4):
                        @pl.when(lax.rem(cnt - 1, 4) == p)
                        def _():
                            pltpu.make_async_copy(
                                rows[p], acc.at[didx.at[cnt - 1]],
                                ssem[p]).wait()

        plsc.subcore_barrier()
        pltpu.sync_copy(acc.at[pl.ds(s * NZ, NZ)],
                        out_hbm.at[c, pl.ds(s * NZ, NZ)])

    return agg


def _agg(table, sidx, didx, w, full=False, hist=False):
    zeros = jnp.zeros((NZ, w), jnp.float32)
    if hist:
        return _make_agg(w, full, hist)(table, didx, zeros)
    return _make_agg(w, full, hist)(table, sidx, didx, zeros)


def _row_spec(w):
    return pl.BlockSpec((RB, w), lambda i: (i, 0))


def _fix_spec(shape):
    return pl.BlockSpec(shape, lambda i: (0,) * len(shape))


def _pair_spec(w):
    return pl.BlockSpec((2, RB, w), lambda i: (0, i, 0))


def _s0_body(dp_ref, x_ref, t1_ref, dinv8_ref):
    deg = dp_ref[0, :, 0:1] + dp_ref[1, :, 0:1] + 1.0
    di = lax.rsqrt(deg)
    dinv8_ref[...] = jnp.broadcast_to(di, (RB, 8))
    t = di * x_ref[...]
    t1_ref[...] = jnp.concatenate([t, jnp.zeros((RB, 2), jnp.float32)], axis=1)


def _stage0(dp, x):
    return pl.pallas_call(
        _s0_body,
        grid=(GRID,),
        in_specs=[_pair_spec(8), _row_spec(6)],
        out_specs=[_row_spec(8), _row_spec(8)],
        out_shape=[jax.ShapeDtypeStruct((N, 8), jnp.float32),
                   jax.ShapeDtypeStruct((N, 8), jnp.float32)],
    )(dp, x)


def _layer_body(a_ref, tp_ref, dinv8_ref, w_ref, b_ref, out_ref):
    di = dinv8_ref[:, 0:1]
    u = (a_ref[0] + a_ref[1] + tp_ref[...]) * di
    h = jnp.dot(u, w_ref[...], preferred_element_type=jnp.float32) + b_ref[...]
    t = jnp.maximum(h, 0.0) * di
    if out_ref.shape[0] == 2:
        hw = out_ref.shape[2]
        out_ref[0] = t[:, :hw]
        out_ref[1] = t[:, hw:]
    else:
        out_ref[...] = t


def _layer(a, tp, dinv8, w_mat, b, wi, wo, split=False):
    if split:
        out_spec = pl.BlockSpec((2, RB, wo // 2), lambda i: (0, i, 0))
        out_shape = jax.ShapeDtypeStruct((2, N, wo // 2), jnp.float32)
    else:
        out_spec = _row_spec(wo)
        out_shape = jax.ShapeDtypeStruct((N, wo), jnp.float32)
    return pl.pallas_call(
        _layer_body,
        grid=(GRID,),
        in_specs=[_pair_spec(wi), _row_spec(wi), _row_spec(8),
                  _fix_spec((wi, wo)), _fix_spec((wo,))],
        out_specs=out_spec,
        out_shape=out_shape,
    )(a, tp, dinv8, w_mat, b)


def _s3_body(a_ref, t3_ref, dinv8_ref, w3_ref, b3_ref, w4_ref, t4_ref):
    di = dinv8_ref[:, 0:1]
    a = jnp.concatenate([a_ref[0], a_ref[1]], axis=1)
    t3 = jnp.concatenate([t3_ref[0], t3_ref[1]], axis=1)
    u = (a + t3) * di
    h = jnp.dot(u, w3_ref[...], preferred_element_type=jnp.float32) + b3_ref[...]
    h = jnp.maximum(h, 0.0)
    z = jnp.dot(h, w4_ref[...], preferred_element_type=jnp.float32) * di
    t4_ref[...] = jnp.concatenate([z, jnp.zeros((RB, 6), jnp.float32)], axis=1)


def _stage3(a, t3, dinv8, w3, b3, w4):
    return pl.pallas_call(
        _s3_body,
        grid=(GRID,),
        in_specs=[_pair_spec(32), _pair_spec(32), _row_spec(8),
                  _fix_spec((64, 128)), _fix_spec((128,)), _fix_spec((128, 2))],
        out_specs=_row_spec(8),
        out_shape=jax.ShapeDtypeStruct((N, 8), jnp.float32),
    )(a, t3, dinv8, w3, b3, w4)


def _s4_body(a_ref, t4_ref, dinv8_ref, b4_ref, out_ref):
    di = dinv8_ref[:, 0:1]
    v = (a_ref[0] + a_ref[1] + t4_ref[...])[:, 0:2] * di + b4_ref[...]
    m = jnp.max(v, axis=1, keepdims=True)
    e = jnp.exp(v - m)
    out_ref[...] = (v - m) - jnp.log(jnp.sum(e, axis=1, keepdims=True))


def _stage4(a, t4, dinv8, b4):
    return pl.pallas_call(
        _s4_body,
        grid=(GRID,),
        in_specs=[_pair_spec(8), _row_spec(8), _row_spec(8), _fix_spec((2,))],
        out_specs=_row_spec(2),
        out_shape=jax.ShapeDtypeStruct((N, 2), jnp.float32),
    )(a, t4, dinv8, b4)


def kernel(x, edge_index, W1, b1, W2, b2, W3, b3, W4, b4):
    src = edge_index[0].astype(jnp.int32)
    dst = edge_index[1].astype(jnp.int32)

    sp = jnp.pad(src.reshape(NCH, B), ((0, NCH_PAD - NCH), (0, 0)))
    sidx_all = jnp.stack([sp, sp + N])
    didx_all = jnp.pad(dst.reshape(NCH, B),
                       ((0, NCH_PAD - NCH), (0, 0)))[None]
    sidx_all, didx_all = lax.optimization_barrier((sidx_all, didx_all))

    dp = _agg(jnp.ones((B, 8), jnp.float32), sidx_all, didx_all, 8, hist=True)
    t1, dinv8 = _stage0(dp, x)

    a1 = _agg(t1, sidx_all, didx_all, 8)
    t2 = _layer(a1, t1, dinv8, jnp.pad(W1, ((0, 2), (0, 0))), b1, 8, 32)

    a2 = _agg(t2, sidx_all, didx_all, 32)
    t3s = _layer(a2, t2, dinv8, W2, b2, 32, 64, split=True)

    a3 = _agg(t3s.reshape(2 * N, 32), sidx_all, didx_all, 32, full=True)
    t4 = _stage3(a3, t3s, dinv8, W3, b3, W4)

    a4 = _agg(t4, sidx_all, didx_all, 8)
    return _stage4(a4, t4, dinv8, b4)


# speedup vs baseline: 36.4746x; 1.0164x over previous
import functools

import jax
import jax.numpy as jnp
from jax import lax
from jax.experimental import pallas as pl
from jax.experimental.pallas import tpu as pltpu
from jax.experimental.pallas import tpu_sc as plsc

N = 50000
E = 800000
B = 128
NSUB = 16
NCORE = 2
N_ACC = 50048
NZ = N_ACC // NSUB
NCH = E // B
CH_HALF = -(-(NCH // NCORE) // NSUB)
CH_FULL = -(-NCH // NSUB)
G = 48
NCH_PAD = 6400

RB = 5000
GRID = N // RB


@functools.lru_cache(maxsize=None)
def _make_agg(w, full, hist=False):
    mesh = plsc.VectorSubcoreMesh(core_axis_name="c", subcore_axis_name="s")
    per = NCH // NCORE if not full else NCH
    lo = per // NSUB
    extra = per - lo * NSUB
    g = 128 if w <= 8 else G
    ngrp = -(-(lo + 1) // g)

    @functools.partial(
        pl.kernel,
        out_type=jax.ShapeDtypeStruct((NCORE, N_ACC, w), jnp.float32),
        mesh=mesh,
        scratch_types=[
            pltpu.VMEM((g, B), jnp.int32),
            pltpu.VMEM((g, B), jnp.int32),
            [pltpu.VMEM((B, w), jnp.float32)] * 4,
            pltpu.VMEM_SHARED((N_ACC, w), jnp.float32),
            [pltpu.SemaphoreType.DMA] * 4,
            [pltpu.SemaphoreType.DMA] * 4,
        ],
        compiler_params=pltpu.CompilerParams(use_tc_tiling_on_sc=False),
    )
    def agg(*refs):
        if hist:
            table_hbm, didx_hbm, zeros_hbm, out_hbm, \
                sidx, didx, rows, acc, gsem, ssem = refs
            sidx_hbm = None
        else:
            table_hbm, sidx_hbm, didx_hbm, zeros_hbm, out_hbm, \
                sidx, didx, rows, acc, gsem, ssem = refs
        c = lax.axis_index("c")
        s = lax.axis_index("s")
        nch = lo + jnp.where(s < extra, 1, 0)
        base = s * lo + jnp.minimum(s, extra) + (0 if full else c * per)
        csel = c if full else 0

        pltpu.sync_copy(zeros_hbm, acc.at[pl.ds(s * NZ, NZ)])
        if hist:
            pltpu.sync_copy(table_hbm, rows[0])
        plsc.subcore_barrier()

        def step(k, cnt, p):
            q = (p + 3) % 4
            pltpu.make_async_copy(
                table_hbm.at[pl.ds(0, B)], rows[p], gsem[p]).wait()
            pltpu.async_copy(rows[p], acc.at[didx.at[k]], ssem[p], add=True)

            @pl.when(k >= 1)
            def _():
                pltpu.make_async_copy(
                    rows[q], acc.at[didx.at[k - 1]], ssem[q]).wait()

            @pl.when(k + 3 < cnt)
            def _():
                pltpu.async_copy(
                    table_hbm.at[sidx.at[k + 3]], rows[q], gsem[q])

        def hist_step(k, p):
            pltpu.async_copy(rows[0], acc.at[didx.at[k]], ssem[p], add=True)

            @pl.when(k >= 3)
            def _():
                pltpu.make_async_copy(
                    rows[0], acc.at[didx.at[k - 3]], ssem[(p + 1) % 4]).wait()

        @pl.loop(0, ngrp)
        def _(gi):
            cnt = jnp.minimum(g, nch - gi * g)

            @pl.when(cnt > 0)
            def _():
                pltpu.sync_copy(didx_hbm.at[0, pl.ds(base + gi * g, g)], didx)
                if hist:
                    @pl.loop(0, cnt)
                    def _(k):
                        for p in range(4):
                            @pl.when(lax.rem(k, 4) == p)
                            def _():
                                hist_step(k, p)

                    @pl.loop(jnp.maximum(cnt - 3, 0), cnt)
                    def _(r):
                        for p in range(4):
                            @pl.when(lax.rem(r, 4) == p)
                            def _():
                                pltpu.make_async_copy(
                                    rows[0], acc.at[didx.at[r]],
                                    ssem[p]).wait()
                else:
                    pltpu.sync_copy(
                        sidx_hbm.at[csel, pl.ds(base + gi * g, g)], sidx)
                    for r in range(3):
                        @pl.when(r < cnt)
                        def _():
                            pltpu.async_copy(
                                table_hbm.at[sidx.at[r]], rows[r], gsem[r])

                    @pl.loop(0, cnt)
                    def _(k):
                        for p in range(4):
                            @pl.when(lax.rem(k, 4) == p)
                            def _():
                                step(k, cnt, p)

                    for p in range(4):
                        @pl.when(lax.rem(cnt - 1, 4) == p)
                        def _():
                            pltpu.make_async_copy(
                                rows[p], acc.at[didx.at[cnt - 1]],
                                ssem[p]).wait()

        plsc.subcore_barrier()
        pltpu.sync_copy(acc.at[pl.ds(s * NZ, NZ)],
                        out_hbm.at[c, pl.ds(s * NZ, NZ)])

    return agg


def _agg(table, sidx, didx, w, full=False, hist=False):
    zeros = jnp.zeros((NZ, w), jnp.float32)
    if hist:
        return _make_agg(w, full, hist)(table, didx, zeros)
    return _make_agg(w, full, hist)(table, sidx, didx, zeros)


def _row_spec(w):
    return pl.BlockSpec((RB, w), lambda i: (i, 0))


def _fix_spec(shape):
    return pl.BlockSpec(shape, lambda i: (0,) * len(shape))


def _pair_spec(w):
    return pl.BlockSpec((2, RB, w), lambda i: (0, i, 0))


def _s0_body(dp_ref, x_ref, t1_ref, dinv8_ref):
    deg = dp_ref[0, :, 0:1] + dp_ref[1, :, 0:1] + 1.0
    di = lax.rsqrt(deg)
    dinv8_ref[...] = jnp.broadcast_to(di, (RB, 8))
    t = di * x_ref[...]
    t1_ref[...] = jnp.concatenate([t, jnp.zeros((RB, 2), jnp.float32)], axis=1)


def _stage0(dp, x):
    return pl.pallas_call(
        _s0_body,
        grid=(GRID,),
        in_specs=[_pair_spec(8), _row_spec(6)],
        out_specs=[_row_spec(8), _row_spec(8)],
        out_shape=[jax.ShapeDtypeStruct((N, 8), jnp.float32),
                   jax.ShapeDtypeStruct((N, 8), jnp.float32)],
    )(dp, x)


def _layer_body(a_ref, tp_ref, dinv8_ref, w_ref, b_ref, out_ref):
    di = dinv8_ref[:, 0:1]
    u = (a_ref[0] + a_ref[1] + tp_ref[...]) * di
    h = jnp.dot(u, w_ref[...], preferred_element_type=jnp.float32) + b_ref[...]
    t = jnp.maximum(h, 0.0) * di
    if out_ref.shape[0] == 2:
        hw = out_ref.shape[2]
        out_ref[0] = t[:, :hw]
        out_ref[1] = t[:, hw:]
    else:
        out_ref[...] = t


def _layer(a, tp, dinv8, w_mat, b, wi, wo, split=False):
    if split:
        out_spec = pl.BlockSpec((2, RB, wo // 2), lambda i: (0, i, 0))
        out_shape = jax.ShapeDtypeStruct((2, N, wo // 2), jnp.float32)
    else:
        out_spec = _row_spec(wo)
        out_shape = jax.ShapeDtypeStruct((N, wo), jnp.float32)
    return pl.pallas_call(
        _layer_body,
        grid=(GRID,),
        in_specs=[_pair_spec(wi), _row_spec(wi), _row_spec(8),
                  _fix_spec((wi, wo)), _fix_spec((wo,))],
        out_specs=out_spec,
        out_shape=out_shape,
    )(a, tp, dinv8, w_mat, b)


def _s3_body(a_ref, t3_ref, dinv8_ref, w3_ref, b3_ref, w4_ref, t4_ref):
    di = dinv8_ref[:, 0:1]
    a = jnp.concatenate([a_ref[0], a_ref[1]], axis=1)
    t3 = jnp.concatenate([t3_ref[0], t3_ref[1]], axis=1)
    u = (a + t3) * di
    h = jnp.dot(u, w3_ref[...], preferred_element_type=jnp.float32) + b3_ref[...]
    h = jnp.maximum(h, 0.0)
    z = jnp.dot(h, w4_ref[...], preferred_element_type=jnp.float32) * di
    t4_ref[...] = jnp.concatenate([z, jnp.zeros((RB, 6), jnp.float32)], axis=1)


def _stage3(a, t3, dinv8, w3, b3, w4):
    return pl.pallas_call(
        _s3_body,
        grid=(GRID,),
        in_specs=[_pair_spec(32), _pair_spec(32), _row_spec(8),
                  _fix_spec((64, 128)), _fix_spec((128,)), _fix_spec((128, 2))],
        out_specs=_row_spec(8),
        out_shape=jax.ShapeDtypeStruct((N, 8), jnp.float32),
    )(a, t3, dinv8, w3, b3, w4)


def _s4_body(a_ref, t4_ref, dinv8_ref, b4_ref, out_ref):
    di = dinv8_ref[:, 0:1]
    v = (a_ref[0] + a_ref[1] + t4_ref[...])[:, 0:2] * di + b4_ref[...]
    m = jnp.max(v, axis=1, keepdims=True)
    e = jnp.exp(v - m)
    out_ref[...] = (v - m) - jnp.log(jnp.sum(e, axis=1, keepdims=True))


def _stage4(a, t4, dinv8, b4):
    return pl.pallas_call(
        _s4_body,
        grid=(GRID,),
        in_specs=[_pair_spec(8), _row_spec(8), _row_spec(8), _fix_spec((2,))],
        out_specs=_row_spec(2),
        out_shape=jax.ShapeDtypeStruct((N, 2), jnp.float32),
    )(a, t4, dinv8, b4)


def kernel(x, edge_index, W1, b1, W2, b2, W3, b3, W4, b4):
    src = edge_index[0].astype(jnp.int32)
    dst = edge_index[1].astype(jnp.int32)

    sp = jnp.pad(src.reshape(NCH, B), ((0, NCH_PAD - NCH), (0, 0)))
    sidx_all = jnp.stack([sp, sp + N])
    didx_all = jnp.pad(dst.reshape(NCH, B),
                       ((0, NCH_PAD - NCH), (0, 0)))[None]
    sidx_all, didx_all = lax.optimization_barrier((sidx_all, didx_all))

    dp = _agg(jnp.ones((B, 8), jnp.float32), sidx_all, didx_all, 8, hist=True)
    t1, dinv8 = _stage0(dp, x)

    a1 = _agg(t1, sidx_all, didx_all, 8)
    t2 = _layer(a1, t1, dinv8, jnp.pad(W1, ((0, 2), (0, 0))), b1, 8, 32)

    a2 = _agg(t2, sidx_all, didx_all, 32)
    t3s = _layer(a2, t2, dinv8, W2, b2, 32, 64, split=True)

    a3 = _agg(t3s.reshape(2 * N, 32), sidx_all, didx_all, 32, full=True)
    t4 = _stage3(a3, t3s, dinv8, W3, b3, W4)

    a4 = _agg(t4, sidx_all, didx_all, 8)
    return _stage4(a4, t4, dinv8, b4)
